# baseline (speedup 1.0000x reference)
import functools

import jax
import jax.numpy as jnp
from jax import lax
from jax.experimental import pallas as pl
from jax.experimental.pallas import tpu as pltpu
from jax.experimental.pallas import tpu_sc as plsc

_LANES = 16


def kernel(user_ix, item_ix, user_emb, item_emb):
    B = user_ix.shape[0]
    V, F = user_emb.shape
    info = plsc.get_sparse_core_info()
    NC, NS = info.num_cores, info.num_subcores
    NW = NC * NS
    C = B // NW
    G = F // 8
    nblk = C // _LANES
    assert B == NW * C and F == 8 * G and C == nblk * _LANES and nblk % 2 == 0

    ut3 = user_emb.T.reshape(G, 8, V)
    it3 = item_emb.T.reshape(G, 8, V)
    uixf = user_ix.reshape(B)
    iixf = item_ix.reshape(B)

    mesh = plsc.VectorSubcoreMesh(core_axis_name="c", subcore_axis_name="s")

    BLK = _LANES * _LANES

    @functools.partial(
        pl.kernel,
        mesh=mesh,
        out_type=jax.ShapeDtypeStruct((F, B), jnp.float32),
        compiler_params=pltpu.CompilerParams(needs_layout_passes=False),
        scratch_types=[
            pltpu.VMEM((C,), jnp.int32),
            pltpu.VMEM((C,), jnp.int32),
            pltpu.VMEM((G, 8, BLK), jnp.float32),
            pltpu.VMEM((G, 8, BLK), jnp.float32),
            pltpu.VMEM((G, 8, C), jnp.float32),
            pltpu.SemaphoreType.DMA,
            pltpu.SemaphoreType.DMA,
        ],
    )
    def run(ut_hbm, it_hbm, uix_hbm, iix_hbm, out_hbm,
            uidx_v, iidx_v, ublk0, iblk0, prod_v, sem_u0, sem_i0):
        wid = lax.axis_index("s") * NC + lax.axis_index("c")
        base = wid * C
        pltpu.sync_copy(uix_hbm.at[pl.ds(base, C)], uidx_v)
        pltpu.sync_copy(iix_hbm.at[pl.ds(base, C)], iidx_v)

        iota = lax.iota(jnp.int32, _LANES)

        def fire(bb, tbl_hbm, idx_ref, blk, sem):
            sl = pl.ds(bb * _LANES, _LANES)
            al0 = idx_ref[sl] & jnp.int32(-_LANES)

            def fire4(j2, al):
                for jj in range(4):
                    dst = pl.ds(j2 * (4 * _LANES) + jj * _LANES, _LANES)
                    b16 = pl.multiple_of(al[jj], _LANES)
                    for g in range(G):
                        pltpu.async_copy(
                            tbl_hbm.at[g, :, pl.ds(b16, _LANES)],
                            blk.at[g, :, dst], sem)
                rot = ((iota + 4) & (_LANES - 1))[:, None]
                dn = lax.GatherDimensionNumbers(
                    offset_dims=(), collapsed_slice_dims=(0,),
                    start_index_map=(0,))
                return lax.gather(
                    al, rot, dimension_numbers=dn, slice_sizes=(1,),
                    mode=lax.GatherScatterMode.PROMISE_IN_BOUNDS)

            lax.fori_loop(0, _LANES // 4, fire4, al0)

        def drain_extract(bb, tbl_hbm, idx_ref, blk, sem, mul):
            pltpu.make_async_copy(
                tbl_hbm.at[:, :, pl.ds(0, BLK)], blk, sem).wait()
            sl = pl.ds(bb * _LANES, _LANES)
            lane = iota * _LANES + (idx_ref[sl] & (_LANES - 1))
            for g in range(G):
                gg = jnp.full((_LANES,), g, jnp.int32)
                for s in range(8):
                    ss = jnp.full((_LANES,), s, jnp.int32)
                    v16 = plsc.load_gather(blk, [gg, ss, lane])
                    if mul:
                        prod_v[g, s, sl] = prod_v[g, s, sl] * v16
                    else:
                        prod_v[g, s, sl] = v16

        def ublock(bb, _):
            fire(bb, ut_hbm, uidx_v, ublk0, sem_u0)
            drain_extract(bb, ut_hbm, uidx_v, ublk0, sem_u0, False)
            return 0

        def iblock(bb, _):
            fire(bb, it_hbm, iidx_v, iblk0, sem_i0)
            drain_extract(bb, it_hbm, iidx_v, iblk0, sem_i0, True)
            return 0

        lax.fori_loop(0, nblk, ublock, 0)
        lax.fori_loop(0, nblk, iblock, 0)

        for g in range(G):
            for s in range(8):
                pltpu.sync_copy(
                    prod_v.at[g, s], out_hbm.at[8 * g + s, pl.ds(base, C)])

    out = run(ut3, it3, uixf, iixf)
    return out.T


# speedup vs baseline: 1.0043x; 1.0043x over previous
import functools

import jax
import jax.numpy as jnp
from jax import lax
from jax.experimental import pallas as pl
from jax.experimental.pallas import tpu as pltpu
from jax.experimental.pallas import tpu_sc as plsc

_LANES = 16


def kernel(user_ix, item_ix, user_emb, item_emb):
    B = user_ix.shape[0]
    V, F = user_emb.shape
    info = plsc.get_sparse_core_info()
    NC, NS = info.num_cores, info.num_subcores
    NW = NC * NS
    C = B // NW
    G = F // 8
    nblk = C // _LANES
    assert B == NW * C and F == 8 * G and C == nblk * _LANES and nblk % 2 == 0

    ut3 = user_emb.T.reshape(G, 8, V)
    it3 = item_emb.T.reshape(G, 8, V)
    uixf = user_ix.reshape(B)
    iixf = item_ix.reshape(B)

    mesh = plsc.VectorSubcoreMesh(core_axis_name="c", subcore_axis_name="s")

    BLK = _LANES * _LANES

    @functools.partial(
        pl.kernel,
        mesh=mesh,
        out_type=jax.ShapeDtypeStruct((F, B), jnp.float32),
        compiler_params=pltpu.CompilerParams(needs_layout_passes=False),
        scratch_types=[
            pltpu.VMEM((C,), jnp.int32),
            pltpu.VMEM((C,), jnp.int32),
            pltpu.VMEM((G, 8, BLK), jnp.float32),
            pltpu.VMEM((G, 8, BLK), jnp.float32),
            pltpu.VMEM((G, 8, C), jnp.float32),
            pltpu.SemaphoreType.DMA,
            pltpu.SemaphoreType.DMA,
        ],
    )
    def run(ut_hbm, it_hbm, uix_hbm, iix_hbm, out_hbm,
            uidx_v, iidx_v, ublk0, iblk0, prod_v, sem_u0, sem_i0):
        wid = lax.axis_index("s") * NC + lax.axis_index("c")
        base = wid * C
        pltpu.sync_copy(uix_hbm.at[pl.ds(base, C)], uidx_v)
        pltpu.sync_copy(iix_hbm.at[pl.ds(base, C)], iidx_v)

        iota = lax.iota(jnp.int32, _LANES)

        def fire(bb, ublk, iblk, su, si):
            sl = pl.ds(bb * _LANES, _LANES)
            ual0 = uidx_v[sl] & jnp.int32(-_LANES)
            ial0 = iidx_v[sl] & jnp.int32(-_LANES)

            def fire4(j2, carry):
                ual, ial = carry
                for jj in range(4):
                    dst = pl.ds(j2 * (4 * _LANES) + jj * _LANES, _LANES)
                    ub = pl.multiple_of(ual[jj], _LANES)
                    ib = pl.multiple_of(ial[jj], _LANES)
                    for g in range(G):
                        pltpu.async_copy(
                            ut_hbm.at[g, :, pl.ds(ub, _LANES)],
                            ublk.at[g, :, dst], su)
                        pltpu.async_copy(
                            it_hbm.at[g, :, pl.ds(ib, _LANES)],
                            iblk.at[g, :, dst], si)
                rot = ((iota + 4) & (_LANES - 1))[:, None]
                dn = lax.GatherDimensionNumbers(
                    offset_dims=(), collapsed_slice_dims=(0,),
                    start_index_map=(0,))
                gather4 = functools.partial(
                    lax.gather, dimension_numbers=dn, slice_sizes=(1,),
                    mode=lax.GatherScatterMode.PROMISE_IN_BOUNDS)
                return (gather4(ual, rot), gather4(ial, rot))

            lax.fori_loop(0, _LANES // 4, fire4, (ual0, ial0))

        def drain_extract(bb, ublk, iblk, su, si):
            pltpu.make_async_copy(
                ut_hbm.at[:, :, pl.ds(0, BLK)], ublk, su).wait()
            pltpu.make_async_copy(
                it_hbm.at[:, :, pl.ds(0, BLK)], iblk, si).wait()
            sl = pl.ds(bb * _LANES, _LANES)
            ulane = iota * _LANES + (uidx_v[sl] & (_LANES - 1))
            ilane = iota * _LANES + (iidx_v[sl] & (_LANES - 1))
            for g in range(G):
                gg = jnp.full((_LANES,), g, jnp.int32)
                for s in range(8):
                    ss = jnp.full((_LANES,), s, jnp.int32)
                    u16 = plsc.load_gather(ublk, [gg, ss, ulane])
                    i16 = plsc.load_gather(iblk, [gg, ss, ilane])
                    prod_v[g, s, sl] = u16 * i16

        def block(bb, _):
            fire(bb, ublk0, iblk0, sem_u0, sem_i0)
            drain_extract(bb, ublk0, iblk0, sem_u0, sem_i0)
            return 0

        lax.fori_loop(0, nblk, block, 0)

        for g in range(G):
            for s in range(8):
                pltpu.sync_copy(
                    prod_v.at[g, s], out_hbm.at[8 * g + s, pl.ds(base, C)])

    out = run(ut3, it3, uixf, iixf)
    return out.T
